# baseline (speedup 1.0000x reference)
import functools

import jax
import jax.numpy as jnp
from jax import lax
from jax.experimental import pallas as pl
from jax.experimental.pallas import tpu as pltpu
from jax.experimental.pallas import tpu_sc as plsc

_L = 16


def kernel(x, seg, tok_table, pos_table, seg_table, gamma, beta):
    B, S = x.shape
    V, D = tok_table.shape
    J = D // _L
    N = B * S
    NC, NS = 2, 16
    NW = NC * NS
    T = N // NW
    C = 64
    G = T // C
    NB = 4

    x_flat = x.reshape(N)
    seg_flat = seg.reshape(N)

    @functools.partial(
        pl.kernel,
        out_type=jax.ShapeDtypeStruct((N, D), jnp.float32),
        mesh=plsc.VectorSubcoreMesh(core_axis_name="c", subcore_axis_name="s"),
        compiler_params=pltpu.CompilerParams(needs_layout_passes=False),
        scratch_types=[
            pltpu.VMEM((S, D), jnp.float32),
            pltpu.VMEM((2, D), jnp.float32),
            pltpu.VMEM((D,), jnp.float32),
            pltpu.VMEM((D,), jnp.float32),
            pltpu.VMEM((T,), jnp.int32),
            pltpu.VMEM((NB, C), jnp.int32),
            pltpu.VMEM((NB, C, D), jnp.float32),
            pltpu.VMEM((C // _L * 2 * _L * _L,), jnp.float32),
            [pltpu.SemaphoreType.DMA] * NB,
            [pltpu.SemaphoreType.DMA] * NB,
        ],
    )
    def run(x_hbm, seg_hbm, tok_hbm, pos_hbm, st_hbm, g_hbm, b_hbm, out_hbm,
            posb, stb, gb, bb, idxall, segb, rows, sqb, gsems, osems):
        wid = lax.axis_index("s") * NC + lax.axis_index("c")
        wbase = wid * T

        pltpu.sync_copy(x_hbm.at[pl.ds(wbase, T)], idxall)
        pltpu.sync_copy(pos_hbm, posb)
        pltpu.sync_copy(st_hbm, stb)
        pltpu.sync_copy(g_hbm, gb)
        pltpu.sync_copy(b_hbm, bb)

        s0 = [stb[0, pl.ds(j * _L, _L)] for j in range(J)]
        s1 = [stb[1, pl.ds(j * _L, _L)] for j in range(J)]
        dsg = [a - b for a, b in zip(s1, s0)]
        gv = [gb[pl.ds(j * _L, _L)] for j in range(J)]
        bv = [bb[pl.ds(j * _L, _L)] for j in range(J)]

        def fold(p, carry):
            for j in range(J):
                posb[p, pl.ds(j * _L, _L)] = posb[p, pl.ds(j * _L, _L)] + s0[j]
            return carry

        lax.fori_loop(0, S, fold, 0)

        def start_chunk(g_, b):
            off = pl.multiple_of(g_ * C, C)
            pltpu.async_copy(
                tok_hbm.at[idxall.at[pl.ds(off, C)]], rows.at[b], gsems[b])
            pltpu.async_copy(
                seg_hbm.at[pl.ds(wbase + off, C)], segb.at[b], gsems[b])

        def wait_chunk(b):
            pltpu.make_async_copy(
                tok_hbm.at[idxall.at[pl.ds(0, C)]], rows.at[b],
                gsems[b]).wait()
            pltpu.make_async_copy(
                seg_hbm.at[pl.ds(wbase, C)], segb.at[b], gsems[b]).wait()

        def start_out(g_, b):
            off = pl.multiple_of(g_ * C, C)
            pltpu.async_copy(
                rows.at[b], out_hbm.at[pl.ds(wbase + off, C)], osems[b])

        def wait_out(b):
            pltpu.make_async_copy(
                rows.at[b], out_hbm.at[pl.ds(wbase, C)], osems[b]).wait()

        def _tree(vs):
            vs = list(vs)
            while len(vs) > 1:
                tail = [vs[-1]] if len(vs) % 2 else []
                vs = [a2 + b2 for a2, b2 in zip(vs[::2], vs[1::2])] + tail
            return vs[0]

        def compute_chunk(g_, b):
            pb = lax.rem(g_ * C, S)
            iota = lax.iota(jnp.int32, _L)

            @plsc.parallel_loop(0, C // _L, unroll=2)
            def group(gi):
                tb = gi * _L
                sfv = segb[b, pl.ds(tb, _L)].astype(jnp.float32)
                for k in range(_L):
                    t = tb + k
                    sf = sfv[k]
                    p = pb + t
                    hs = []
                    for j in range(J):
                        tv = rows[b, t, pl.ds(j * _L, _L)]
                        pv = posb[p, pl.ds(j * _L, _L)]
                        hs.append(tv + pv + sf * dsg[j])
                    sv = _tree(hs)
                    qv = _tree([h * h for h in hs])
                    for j in range(J):
                        rows[b, t, pl.ds(j * _L, _L)] = hs[j]
                    sqb[pl.ds(gi * 2 * _L * _L + k * _L, _L)] = sv
                    sqb[pl.ds(gi * 2 * _L * _L + (_L + k) * _L, _L)] = qv
                scols = []
                qcols = []
                sbase = iota * _L + gi * 2 * _L * _L
                qbase = sbase + _L * _L
                for l in range(_L):
                    scols.append(plsc.load_gather(sqb, [sbase + l]))
                    qcols.append(plsc.load_gather(sqb, [qbase + l]))
                sum_v = _tree(scols)
                q_v = _tree(qcols)
                mean_v = sum_v * (1.0 / D)
                var_v = q_v * (1.0 / D) - mean_v * mean_v
                vb = var_v + 1e-5
                ib = lax.bitcast_convert_type(vb, jnp.int32)
                yi = jnp.full((_L,), 0x5F3759DF, jnp.int32) - (
                    lax.shift_right_arithmetic(
                        ib, jnp.full((_L,), 1, jnp.int32)))
                y = lax.bitcast_convert_type(yi, jnp.float32)
                half = 0.5 * vb
                for _ in range(3):
                    y = y * (1.5 - half * y * y)
                for k in range(_L):
                    t = tb + k
                    m_s = mean_v[k]
                    y_s = y[k]
                    for j in range(J):
                        h = rows[b, t, pl.ds(j * _L, _L)]
                        rows[b, t, pl.ds(j * _L, _L)] = (
                            (h - m_s) * y_s * gv[j] + bv[j])

        start_chunk(0, 0)
        start_chunk(1, 1)

        def outer(gg, carry):
            for b in range(NB):
                g_ = gg * NB + b
                b2 = (b + 2) % NB
                wait_chunk(b)

                @pl.when(g_ >= 2)
                def _():
                    wait_out(b2)

                @pl.when(g_ + 2 < G)
                def _():
                    start_chunk(g_ + 2, b2)

                compute_chunk(g_, b)
                start_out(g_, b)
            return carry

        lax.fori_loop(0, G // NB, outer, 0)
        for k in range(G - 2, G):
            wait_out(k % NB)

    out = run(x_flat, seg_flat, tok_table, pos_table, seg_table, gamma, beta)
    return out.reshape(B, S, D)


# speedup vs baseline: 1.3134x; 1.3134x over previous
import functools

import jax
import jax.numpy as jnp
from jax import lax
from jax.experimental import pallas as pl
from jax.experimental.pallas import tpu as pltpu
from jax.experimental.pallas import tpu_sc as plsc

_L = 16


def kernel(x, seg, tok_table, pos_table, seg_table, gamma, beta):
    B, S = x.shape
    V, D = tok_table.shape
    J = D // _L
    N = B * S
    NC, NS = 2, 16
    NW = NC * NS
    T = N // NW
    C = 64
    G = T // C
    NB = 4

    x_flat = x.reshape(N)
    seg_flat = seg.reshape(N)

    @functools.partial(
        pl.kernel,
        out_type=jax.ShapeDtypeStruct((N, D), jnp.float32),
        mesh=plsc.VectorSubcoreMesh(core_axis_name="c", subcore_axis_name="s"),
        compiler_params=pltpu.CompilerParams(needs_layout_passes=False),
        scratch_types=[
            pltpu.VMEM((S, D), jnp.float32),
            pltpu.VMEM((2, D), jnp.float32),
            pltpu.VMEM((D,), jnp.float32),
            pltpu.VMEM((D,), jnp.float32),
            pltpu.VMEM((T,), jnp.int32),
            pltpu.VMEM((NB, C), jnp.int32),
            pltpu.VMEM((NB, C, D), jnp.float32),
            pltpu.VMEM((C // _L * 2 * _L * _L,), jnp.float32),
            pltpu.SemaphoreType.DMA((NB,)),
            pltpu.SemaphoreType.DMA((NB,)),
        ],
    )
    def run(x_hbm, seg_hbm, tok_hbm, pos_hbm, st_hbm, g_hbm, b_hbm, out_hbm,
            posb, stb, gb, bb, idxall, segb, rows, sqb, gsems, osems):
        wid = lax.axis_index("s") * NC + lax.axis_index("c")
        wbase = wid * T

        pltpu.sync_copy(x_hbm.at[pl.ds(wbase, T)], idxall)
        pltpu.sync_copy(pos_hbm, posb)
        pltpu.sync_copy(st_hbm, stb)
        pltpu.sync_copy(g_hbm, gb)
        pltpu.sync_copy(b_hbm, bb)

        s0 = [stb[0, pl.ds(j * _L, _L)] for j in range(J)]
        s1 = [stb[1, pl.ds(j * _L, _L)] for j in range(J)]
        dsg = [a - b for a, b in zip(s1, s0)]
        gv = [gb[pl.ds(j * _L, _L)] for j in range(J)]
        bv = [bb[pl.ds(j * _L, _L)] for j in range(J)]

        def fold(p, carry):
            for j in range(J):
                posb[p, pl.ds(j * _L, _L)] = posb[p, pl.ds(j * _L, _L)] + s0[j]
            return carry

        lax.fori_loop(0, S, fold, 0)

        def start_chunk(g_, b):
            off = pl.multiple_of(g_ * C, C)
            pltpu.async_copy(
                tok_hbm.at[idxall.at[pl.ds(off, C)]], rows.at[b],
                gsems.at[b])
            pltpu.async_copy(
                seg_hbm.at[pl.ds(wbase + off, C)], segb.at[b], gsems.at[b])

        def wait_chunk(b):
            pltpu.make_async_copy(
                tok_hbm.at[idxall.at[pl.ds(0, C)]], rows.at[b],
                gsems.at[b]).wait()
            pltpu.make_async_copy(
                seg_hbm.at[pl.ds(wbase, C)], segb.at[b], gsems.at[b]).wait()

        def start_out(g_, b):
            off = pl.multiple_of(g_ * C, C)
            pltpu.async_copy(
                rows.at[b], out_hbm.at[pl.ds(wbase + off, C)], osems.at[b])

        def wait_out(b):
            pltpu.make_async_copy(
                rows.at[b], out_hbm.at[pl.ds(wbase, C)], osems.at[b]).wait()

        def _tree(vs):
            vs = list(vs)
            while len(vs) > 1:
                tail = [vs[-1]] if len(vs) % 2 else []
                vs = [a2 + b2 for a2, b2 in zip(vs[::2], vs[1::2])] + tail
            return vs[0]

        def compute_chunk(g_, b):
            pb = lax.rem(g_ * C, S)
            iota = lax.iota(jnp.int32, _L)

            @plsc.parallel_loop(0, C // _L)
            def group(gi):
                tb = gi * _L
                sfv = segb[b, pl.ds(tb, _L)].astype(jnp.float32)
                for k in range(_L):
                    t = tb + k
                    sf = sfv[k]
                    p = pb + t
                    hs = []
                    for j in range(J):
                        tv = rows[b, t, pl.ds(j * _L, _L)]
                        pv = posb[p, pl.ds(j * _L, _L)]
                        hs.append(tv + pv + sf * dsg[j])
                    sv = _tree(hs)
                    qv = _tree([h * h for h in hs])
                    for j in range(J):
                        rows[b, t, pl.ds(j * _L, _L)] = hs[j]
                    sqb[pl.ds(gi * 2 * _L * _L + k * _L, _L)] = sv
                    sqb[pl.ds(gi * 2 * _L * _L + (_L + k) * _L, _L)] = qv
                scols = []
                qcols = []
                sbase = iota * _L + gi * 2 * _L * _L
                qbase = sbase + _L * _L
                for l in range(_L):
                    scols.append(plsc.load_gather(sqb, [sbase + l]))
                    qcols.append(plsc.load_gather(sqb, [qbase + l]))
                sum_v = _tree(scols)
                q_v = _tree(qcols)
                mean_v = sum_v * (1.0 / D)
                var_v = q_v * (1.0 / D) - mean_v * mean_v
                vb = var_v + 1e-5
                ib = lax.bitcast_convert_type(vb, jnp.int32)
                yi = jnp.full((_L,), 0x5F3759DF, jnp.int32) - (
                    lax.shift_right_arithmetic(
                        ib, jnp.full((_L,), 1, jnp.int32)))
                y = lax.bitcast_convert_type(yi, jnp.float32)
                half = 0.5 * vb
                for _ in range(3):
                    y = y * (1.5 - half * y * y)
                for k in range(_L):
                    t = tb + k
                    m_s = mean_v[k]
                    y_s = y[k]
                    for j in range(J):
                        h = rows[b, t, pl.ds(j * _L, _L)]
                        rows[b, t, pl.ds(j * _L, _L)] = (
                            (h - m_s) * y_s * gv[j] + bv[j])

        start_chunk(0, 0)
        start_chunk(1, 1)

        def outer(g_, carry):
            b = lax.rem(g_, NB)
            b2 = lax.rem(g_ + 2, NB)
            wait_chunk(b)

            @pl.when(g_ >= 2)
            def _():
                wait_out(b2)

            @pl.when(g_ + 2 < G)
            def _():
                start_chunk(g_ + 2, b2)

            compute_chunk(g_, b)
            start_out(g_, b)
            return carry

        lax.fori_loop(0, G, outer, 0)
        for k in range(G - 2, G):
            wait_out(k % NB)

    out = run(x_flat, seg_flat, tok_table, pos_table, seg_table, gamma, beta)
    return out.reshape(B, S, D)


# speedup vs baseline: 2.5033x; 1.9059x over previous
import functools

import jax
import jax.numpy as jnp
from jax import lax
from jax.experimental import pallas as pl
from jax.experimental.pallas import tpu as pltpu
from jax.experimental.pallas import tpu_sc as plsc

_L = 16


def kernel(x, seg, tok_table, pos_table, seg_table, gamma, beta):
    B, S = x.shape
    V, D = tok_table.shape
    J = D // _L
    N = B * S
    NC, NS = 2, 16
    NW = NC * NS
    T = N // NW
    C = 64
    G = T // C
    NB = 2

    x_flat = x.reshape(N)
    seg_flat = seg.reshape(N)

    @functools.partial(
        pl.kernel,
        out_type=jax.ShapeDtypeStruct((N, D), jnp.float32),
        mesh=plsc.VectorSubcoreMesh(core_axis_name="c", subcore_axis_name="s"),
        compiler_params=pltpu.CompilerParams(needs_layout_passes=False),
        scratch_types=[
            pltpu.VMEM((S, D), jnp.float32),
            pltpu.VMEM((2, D), jnp.float32),
            pltpu.VMEM((D,), jnp.float32),
            pltpu.VMEM((D,), jnp.float32),
            pltpu.VMEM((T,), jnp.int32),
            pltpu.VMEM((NB, C), jnp.int32),
            pltpu.VMEM((NB, C, D), jnp.float32),
            pltpu.VMEM((NB, C, D), jnp.float32),
            pltpu.VMEM((C // _L * 2 * _L * _L,), jnp.float32),
            [pltpu.SemaphoreType.DMA] * NB,
            [pltpu.SemaphoreType.DMA] * NB,
        ],
    )
    def run(x_hbm, seg_hbm, tok_hbm, pos_hbm, st_hbm, g_hbm, b_hbm, out_hbm,
            posb, stb, gb, bb, idxall, segb, rows, outs, sqb, gsems, osems):
        wid = lax.axis_index("s") * NC + lax.axis_index("c")
        wbase = wid * T

        pltpu.sync_copy(x_hbm.at[pl.ds(wbase, T)], idxall)
        pltpu.sync_copy(pos_hbm, posb)
        pltpu.sync_copy(st_hbm, stb)
        pltpu.sync_copy(g_hbm, gb)
        pltpu.sync_copy(b_hbm, bb)

        s0 = [stb[0, pl.ds(j * _L, _L)] for j in range(J)]
        s1 = [stb[1, pl.ds(j * _L, _L)] for j in range(J)]
        dsg = [a - b for a, b in zip(s1, s0)]
        gv = [gb[pl.ds(j * _L, _L)] for j in range(J)]
        bv = [bb[pl.ds(j * _L, _L)] for j in range(J)]

        def fold(p, carry):
            for j in range(J):
                posb[p, pl.ds(j * _L, _L)] = posb[p, pl.ds(j * _L, _L)] + s0[j]
            return carry

        lax.fori_loop(0, S, fold, 0)

        def start_chunk(g_, b):
            off = pl.multiple_of(g_ * C, C)
            pltpu.async_copy(
                tok_hbm.at[idxall.at[pl.ds(off, C)]], rows.at[b], gsems[b])
            pltpu.async_copy(
                seg_hbm.at[pl.ds(wbase + off, C)], segb.at[b], gsems[b])

        def wait_chunk(b):
            pltpu.make_async_copy(
                tok_hbm.at[idxall.at[pl.ds(0, C)]], rows.at[b],
                gsems[b]).wait()
            pltpu.make_async_copy(
                seg_hbm.at[pl.ds(wbase, C)], segb.at[b], gsems[b]).wait()

        def start_out(g_, b):
            off = pl.multiple_of(g_ * C, C)
            pltpu.async_copy(
                outs.at[b], out_hbm.at[pl.ds(wbase + off, C)], osems[b])

        def wait_out(b):
            pltpu.make_async_copy(
                outs.at[b], out_hbm.at[pl.ds(wbase, C)], osems[b]).wait()

        def _tree(vs):
            vs = list(vs)
            while len(vs) > 1:
                tail = [vs[-1]] if len(vs) % 2 else []
                vs = [a2 + b2 for a2, b2 in zip(vs[::2], vs[1::2])] + tail
            return vs[0]

        def compute_chunk(g_, b):
            pb = lax.rem(g_ * C, S)
            iota = lax.iota(jnp.int32, _L)

            @plsc.parallel_loop(0, C // _L)
            def group(gi):
                tb = gi * _L
                sfv = segb[b, pl.ds(tb, _L)].astype(jnp.float32)
                for k in range(_L):
                    t = tb + k
                    sf = sfv[k]
                    p = pb + t
                    hs = []
                    for j in range(J):
                        tv = rows[b, t, pl.ds(j * _L, _L)]
                        pv = posb[p, pl.ds(j * _L, _L)]
                        hs.append(tv + pv + sf * dsg[j])
                    sv = _tree(hs)
                    qv = _tree([h * h for h in hs])
                    for j in range(J):
                        outs[b, t, pl.ds(j * _L, _L)] = hs[j]
                    sqb[pl.ds(gi * 2 * _L * _L + k * _L, _L)] = sv
                    sqb[pl.ds(gi * 2 * _L * _L + (_L + k) * _L, _L)] = qv
                scols = []
                qcols = []
                sbase = iota * _L + gi * 2 * _L * _L
                qbase = sbase + _L * _L
                for l in range(_L):
                    scols.append(plsc.load_gather(sqb, [sbase + l]))
                    qcols.append(plsc.load_gather(sqb, [qbase + l]))
                sum_v = _tree(scols)
                q_v = _tree(qcols)
                mean_v = sum_v * (1.0 / D)
                var_v = q_v * (1.0 / D) - mean_v * mean_v
                vb = var_v + 1e-5
                ib = lax.bitcast_convert_type(vb, jnp.int32)
                yi = jnp.full((_L,), 0x5F3759DF, jnp.int32) - (
                    lax.shift_right_arithmetic(
                        ib, jnp.full((_L,), 1, jnp.int32)))
                y = lax.bitcast_convert_type(yi, jnp.float32)
                half = 0.5 * vb
                for _ in range(3):
                    y = y * (1.5 - half * y * y)
                for k in range(_L):
                    t = tb + k
                    m_s = mean_v[k]
                    y_s = y[k]
                    for j in range(J):
                        h = outs[b, t, pl.ds(j * _L, _L)]
                        outs[b, t, pl.ds(j * _L, _L)] = (
                            (h - m_s) * y_s * gv[j] + bv[j])

        start_chunk(0, 0)
        start_chunk(1, 1)

        def outer(gg, carry):
            for b in range(NB):
                g_ = gg * NB + b
                wait_chunk(b)

                @pl.when(g_ >= 2)
                def _():
                    wait_out(b)

                compute_chunk(g_, b)
                start_out(g_, b)

                @pl.when(g_ + 2 < G)
                def _():
                    start_chunk(g_ + 2, b)
            return carry

        lax.fori_loop(0, G // NB, outer, 0)
        for k in range(G - 2, G):
            wait_out(k % NB)

    out = run(x_flat, seg_flat, tok_table, pos_table, seg_table, gamma, beta)
    return out.reshape(B, S, D)


# speedup vs baseline: 4.3579x; 1.7409x over previous
import functools

import jax
import jax.numpy as jnp
from jax import lax
from jax.experimental import pallas as pl
from jax.experimental.pallas import tpu as pltpu
from jax.experimental.pallas import tpu_sc as plsc

_L = 16


def kernel(x, seg, tok_table, pos_table, seg_table, gamma, beta):
    B, S = x.shape
    V, D = tok_table.shape
    J = D // _L
    N = B * S
    NC, NS = 2, 16
    NW = NC * NS
    T = N // NW
    C = 64
    G = T // C
    NB = 2

    x_flat = x.reshape(N)
    seg_flat = seg.reshape(N)

    @functools.partial(
        pl.kernel,
        out_type=jax.ShapeDtypeStruct((N, D), jnp.float32),
        mesh=plsc.VectorSubcoreMesh(core_axis_name="c", subcore_axis_name="s"),
        compiler_params=pltpu.CompilerParams(needs_layout_passes=False),
        scratch_types=[
            pltpu.VMEM((S, D), jnp.float32),
            pltpu.VMEM((2, D), jnp.float32),
            pltpu.VMEM((D,), jnp.float32),
            pltpu.VMEM((D,), jnp.float32),
            pltpu.VMEM((T,), jnp.int32),
            pltpu.VMEM((NB, C), jnp.int32),
            pltpu.VMEM((NB, C, D), jnp.float32),
            pltpu.VMEM((NB, C, D), jnp.float32),
            pltpu.VMEM((C // _L * 2 * _L * _L,), jnp.float32),
            pltpu.VMEM((NB, C), jnp.float32),
            pltpu.VMEM((C,), jnp.float32),
            pltpu.VMEM((C,), jnp.float32),
            [pltpu.SemaphoreType.DMA] * NB,
            [pltpu.SemaphoreType.DMA] * NB,
        ],
    )
    def run(x_hbm, seg_hbm, tok_hbm, pos_hbm, st_hbm, g_hbm, b_hbm, out_hbm,
            posb, stb, gb, bb, idxall, segb, rows, outs, sqb, sfb, mb, yb,
            gsems, osems):
        wid = lax.axis_index("s") * NC + lax.axis_index("c")
        wbase = wid * T

        pltpu.sync_copy(x_hbm.at[pl.ds(wbase, T)], idxall)
        pltpu.sync_copy(pos_hbm, posb)
        pltpu.sync_copy(st_hbm, stb)
        pltpu.sync_copy(g_hbm, gb)
        pltpu.sync_copy(b_hbm, bb)

        s0 = [stb[0, pl.ds(j * _L, _L)] for j in range(J)]
        s1 = [stb[1, pl.ds(j * _L, _L)] for j in range(J)]
        dsg = [a - b for a, b in zip(s1, s0)]
        gv = [gb[pl.ds(j * _L, _L)] for j in range(J)]
        bv = [bb[pl.ds(j * _L, _L)] for j in range(J)]

        def fold(p, carry):
            for j in range(J):
                posb[p, pl.ds(j * _L, _L)] = posb[p, pl.ds(j * _L, _L)] + s0[j]
            return carry

        lax.fori_loop(0, S, fold, 0)

        def start_chunk(g_, b):
            off = pl.multiple_of(g_ * C, C)
            pltpu.async_copy(
                tok_hbm.at[idxall.at[pl.ds(off, C)]], rows.at[b], gsems[b])
            pltpu.async_copy(
                seg_hbm.at[pl.ds(wbase + off, C)], segb.at[b], gsems[b])

        def wait_chunk(b):
            pltpu.make_async_copy(
                tok_hbm.at[idxall.at[pl.ds(0, C)]], rows.at[b],
                gsems[b]).wait()
            pltpu.make_async_copy(
                seg_hbm.at[pl.ds(wbase, C)], segb.at[b], gsems[b]).wait()

        def start_out(g_, b):
            off = pl.multiple_of(g_ * C, C)
            pltpu.async_copy(
                outs.at[b], out_hbm.at[pl.ds(wbase + off, C)], osems[b])

        def wait_out(b):
            pltpu.make_async_copy(
                outs.at[b], out_hbm.at[pl.ds(wbase, C)], osems[b]).wait()

        def _tree(vs):
            vs = list(vs)
            while len(vs) > 1:
                tail = [vs[-1]] if len(vs) % 2 else []
                vs = [a2 + b2 for a2, b2 in zip(vs[::2], vs[1::2])] + tail
            return vs[0]

        def compute_chunk(g_, b):
            pb = lax.rem(g_ * C, S)
            iota = lax.iota(jnp.int32, _L)
            zc = iota * 0

            for q in range(C // _L):
                sfb[b, pl.ds(q * _L, _L)] = (
                    segb[b, pl.ds(q * _L, _L)].astype(jnp.float32))

            @plsc.parallel_loop(0, C)
            def pass1(t):
                sf = plsc.load_gather(sfb, [zc + b, zc + t])
                p = pb + t
                hs = []
                for j in range(J):
                    tv = rows[b, t, pl.ds(j * _L, _L)]
                    pv = posb[p, pl.ds(j * _L, _L)]
                    hs.append(tv + pv + sf * dsg[j])
                for j in range(J):
                    outs[b, t, pl.ds(j * _L, _L)] = hs[j]
                sqb[pl.ds(t * _L, _L)] = _tree(hs)
                sqb[pl.ds(C * _L + t * _L, _L)] = _tree(
                    [h * h for h in hs])

            for gi in range(C // _L):
                sbase = iota * _L + gi * _L * _L
                qbase = sbase + C * _L
                scols = []
                qcols = []
                for l in range(_L):
                    scols.append(plsc.load_gather(sqb, [sbase + l]))
                    qcols.append(plsc.load_gather(sqb, [qbase + l]))
                sum_v = _tree(scols)
                q_v = _tree(qcols)
                mean_v = sum_v * (1.0 / D)
                var_v = q_v * (1.0 / D) - mean_v * mean_v
                vb = var_v + 1e-5
                ib = lax.bitcast_convert_type(vb, jnp.int32)
                yi = jnp.full((_L,), 0x5F3759DF, jnp.int32) - (
                    lax.shift_right_arithmetic(
                        ib, jnp.full((_L,), 1, jnp.int32)))
                y = lax.bitcast_convert_type(yi, jnp.float32)
                half = 0.5 * vb
                for _ in range(3):
                    y = y * (1.5 - half * y * y)
                mb[pl.ds(gi * _L, _L)] = mean_v
                yb[pl.ds(gi * _L, _L)] = y

            @plsc.parallel_loop(0, C)
            def pass2(t):
                m_s = plsc.load_gather(mb, [zc + t])
                y_s = plsc.load_gather(yb, [zc + t])
                for j in range(J):
                    h = outs[b, t, pl.ds(j * _L, _L)]
                    outs[b, t, pl.ds(j * _L, _L)] = (
                        (h - m_s) * y_s * gv[j] + bv[j])

        start_chunk(0, 0)
        start_chunk(1, 1)

        def outer(gg, carry):
            for b in range(NB):
                g_ = gg * NB + b
                wait_chunk(b)

                @pl.when(g_ >= 2)
                def _():
                    wait_out(b)

                compute_chunk(g_, b)
                start_out(g_, b)

                @pl.when(g_ + 2 < G)
                def _():
                    start_chunk(g_ + 2, b)
            return carry

        lax.fori_loop(0, G // NB, outer, 0)
        for k in range(G - 2, G):
            wait_out(k % NB)

    out = run(x_flat, seg_flat, tok_table, pos_table, seg_table, gamma, beta)
    return out.reshape(B, S, D)


# speedup vs baseline: 4.5238x; 1.0381x over previous
import functools

import jax
import jax.numpy as jnp
from jax import lax
from jax.experimental import pallas as pl
from jax.experimental.pallas import tpu as pltpu
from jax.experimental.pallas import tpu_sc as plsc

_L = 16


def kernel(x, seg, tok_table, pos_table, seg_table, gamma, beta):
    B, S = x.shape
    V, D = tok_table.shape
    J = D // _L
    N = B * S
    NC, NS = 2, 16
    NW = NC * NS
    T = N // NW
    C = 64
    G = T // C
    NB = 2

    x_flat = x.reshape(N)
    seg_flat = seg.reshape(N)

    @functools.partial(
        pl.kernel,
        out_type=jax.ShapeDtypeStruct((N, D), jnp.float32),
        mesh=plsc.VectorSubcoreMesh(core_axis_name="c", subcore_axis_name="s"),
        compiler_params=pltpu.CompilerParams(needs_layout_passes=False),
        scratch_types=[
            pltpu.VMEM((S, D), jnp.float32),
            pltpu.VMEM((2, D), jnp.float32),
            pltpu.VMEM((D,), jnp.float32),
            pltpu.VMEM((D,), jnp.float32),
            pltpu.VMEM((T,), jnp.int32),
            pltpu.VMEM((NB, C), jnp.int32),
            pltpu.VMEM((NB, C, D), jnp.float32),
            pltpu.VMEM((NB, C, D), jnp.float32),
            pltpu.VMEM((C // _L * 2 * _L * _L,), jnp.float32),
            pltpu.VMEM((NB * C,), jnp.float32),
            pltpu.VMEM((C,), jnp.float32),
            pltpu.VMEM((C,), jnp.float32),
            [pltpu.SemaphoreType.DMA] * NB,
            [pltpu.SemaphoreType.DMA] * NB,
        ],
    )
    def run(x_hbm, seg_hbm, tok_hbm, pos_hbm, st_hbm, g_hbm, b_hbm, out_hbm,
            posb, stb, gb, bb, idxall, segb, rows, outs, sqb, sfb, mb, yb,
            gsems, osems):
        wid = lax.axis_index("s") * NC + lax.axis_index("c")
        wbase = wid * T

        pltpu.sync_copy(x_hbm.at[pl.ds(wbase, T)], idxall)
        pltpu.sync_copy(pos_hbm, posb)
        pltpu.sync_copy(st_hbm, stb)
        pltpu.sync_copy(g_hbm, gb)
        pltpu.sync_copy(b_hbm, bb)

        s0 = [stb[0, pl.ds(j * _L, _L)] for j in range(J)]
        s1 = [stb[1, pl.ds(j * _L, _L)] for j in range(J)]
        dsg = [a - b for a, b in zip(s1, s0)]
        gv = [gb[pl.ds(j * _L, _L)] for j in range(J)]
        bv = [bb[pl.ds(j * _L, _L)] for j in range(J)]

        def fold(p, carry):
            for j in range(J):
                posb[p, pl.ds(j * _L, _L)] = posb[p, pl.ds(j * _L, _L)] + s0[j]
            return carry

        lax.fori_loop(0, S, fold, 0)

        def start_chunk(g_, b):
            off = pl.multiple_of(g_ * C, C)
            pltpu.async_copy(
                tok_hbm.at[idxall.at[pl.ds(off, C)]], rows.at[b], gsems[b])
            pltpu.async_copy(
                seg_hbm.at[pl.ds(wbase + off, C)], segb.at[b], gsems[b])

        def wait_chunk(b):
            pltpu.make_async_copy(
                tok_hbm.at[idxall.at[pl.ds(0, C)]], rows.at[b],
                gsems[b]).wait()
            pltpu.make_async_copy(
                seg_hbm.at[pl.ds(wbase, C)], segb.at[b], gsems[b]).wait()

        def start_out(g_, b):
            off = pl.multiple_of(g_ * C, C)
            pltpu.async_copy(
                outs.at[b], out_hbm.at[pl.ds(wbase + off, C)], osems[b])

        def wait_out(b):
            pltpu.make_async_copy(
                outs.at[b], out_hbm.at[pl.ds(wbase, C)], osems[b]).wait()

        def _tree(vs):
            vs = list(vs)
            while len(vs) > 1:
                tail = [vs[-1]] if len(vs) % 2 else []
                vs = [a2 + b2 for a2, b2 in zip(vs[::2], vs[1::2])] + tail
            return vs[0]

        def compute_chunk(g_, b):
            pb = lax.rem(g_ * C, S)
            iota = lax.iota(jnp.int32, _L)
            zc = iota * 0

            for q in range(C // _L):
                sfb[pl.ds(b * C + q * _L, _L)] = (
                    segb[b, pl.ds(q * _L, _L)].astype(jnp.float32))

            @plsc.parallel_loop(0, C)
            def pass1(t):
                sf = plsc.load_gather(sfb, [zc + (b * C + t)])
                p = pb + t
                hs = []
                for j in range(J):
                    tv = rows[b, t, pl.ds(j * _L, _L)]
                    pv = posb[p, pl.ds(j * _L, _L)]
                    hs.append(tv + pv + sf * dsg[j])
                for j in range(J):
                    outs[b, t, pl.ds(j * _L, _L)] = hs[j]
                sqb[pl.ds(t * _L, _L)] = _tree(hs)
                sqb[pl.ds(C * _L + t * _L, _L)] = _tree(
                    [h * h for h in hs])

            for gi in range(C // _L):
                sbase = iota * _L + gi * _L * _L
                qbase = sbase + C * _L
                scols = []
                qcols = []
                for l in range(_L):
                    scols.append(plsc.load_gather(sqb, [sbase + l]))
                    qcols.append(plsc.load_gather(sqb, [qbase + l]))
                sum_v = _tree(scols)
                q_v = _tree(qcols)
                mean_v = sum_v * (1.0 / D)
                var_v = q_v * (1.0 / D) - mean_v * mean_v
                vb = var_v + 1e-5
                ib = lax.bitcast_convert_type(vb, jnp.int32)
                yi = jnp.full((_L,), 0x5F3759DF, jnp.int32) - (
                    lax.shift_right_arithmetic(
                        ib, jnp.full((_L,), 1, jnp.int32)))
                y = lax.bitcast_convert_type(yi, jnp.float32)
                half = 0.5 * vb
                for _ in range(3):
                    y = y * (1.5 - half * y * y)
                mb[pl.ds(gi * _L, _L)] = mean_v * y
                yb[pl.ds(gi * _L, _L)] = y

            @plsc.parallel_loop(0, C)
            def pass2(t):
                my_s = plsc.load_gather(mb, [zc + t])
                y_s = plsc.load_gather(yb, [zc + t])
                for j in range(J):
                    h = outs[b, t, pl.ds(j * _L, _L)]
                    outs[b, t, pl.ds(j * _L, _L)] = (
                        (h * y_s - my_s) * gv[j] + bv[j])

        start_chunk(0, 0)
        start_chunk(1, 1)

        def outer(gg, carry):
            for b in range(NB):
                g_ = gg * NB + b
                wait_chunk(b)

                @pl.when(g_ >= 2)
                def _():
                    wait_out(b)

                compute_chunk(g_, b)
                start_out(g_, b)

                @pl.when(g_ + 2 < G)
                def _():
                    start_chunk(g_ + 2, b)
            return carry

        lax.fori_loop(0, G // NB, outer, 0)
        for k in range(G - 2, G):
            wait_out(k % NB)

    out = run(x_flat, seg_flat, tok_table, pos_table, seg_table, gamma, beta)
    return out.reshape(B, S, D)
